# baseline (speedup 1.0000x reference)
import functools

import jax
import jax.numpy as jnp
from jax import lax
from jax.experimental import pallas as pl
from jax.experimental.pallas import tpu as pltpu
from jax.experimental.pallas import tpu_sc as plsc

S = 2048
D = 768
E = 8
FF = 4 * D
BLK = 256
NF = 4
FFS = FF // NF
G = 2 * S + E * BLK
NB = G // BLK
NW = 32
A = 2 * S
CHUNK = A // NW
TCH = S // NW


def _router_body(x_ref, rw_ref, rb_ref, poscat_ref, pos0_ref, pos1_ref,
                 wcat_ref, eids_ref, nact_ref, counts_ref):
    x = x_ref[...]
    logits = lax.dot_general(x, rw_ref[...], (((1,), (1,)), ((), ())),
                             preferred_element_type=jnp.float32)
    logits = logits + rb_ref[...]

    m = jnp.max(logits, axis=1, keepdims=True)
    ex = jnp.exp(logits - m)
    p = ex / jnp.sum(ex, axis=1, keepdims=True)

    lane = lax.broadcasted_iota(jnp.int32, (S, E), 1)
    m1 = jnp.max(p, axis=1, keepdims=True)
    i1 = jnp.min(jnp.where(p == m1, lane, E), axis=1, keepdims=True)
    p2 = jnp.where(lane == i1, -1.0, p)
    m2 = jnp.max(p2, axis=1, keepdims=True)
    i2 = jnp.min(jnp.where(p2 == m2, lane, E), axis=1, keepdims=True)

    e2 = jnp.exp(m2 - m1)
    w0 = 1.0 / (1.0 + e2)
    w1 = e2 / (1.0 + e2)

    oh0 = (lane == i1).astype(jnp.float32)
    oh1 = (lane == i2).astype(jnp.float32)

    CH = 128
    r_i = lax.broadcasted_iota(jnp.int32, (CH, CH), 0)
    c_i = lax.broadcasted_iota(jnp.int32, (CH, CH), 1)
    L = (r_i >= c_i).astype(jnp.float32)

    def chunked_cumsum(oh):
        outs = []
        base = jnp.zeros((1, E), jnp.float32)
        for c in range(S // CH):
            blk = oh[c * CH:(c + 1) * CH, :]
            cc = lax.dot_general(L, blk, (((1,), (0,)), ((), ())),
                                 preferred_element_type=jnp.float32) + base
            outs.append(cc)
            base = cc[CH - 1:CH, :]
        return jnp.concatenate(outs, axis=0), base

    C0, cnt0 = chunked_cumsum(oh0)
    C1, cnt1 = chunked_cumsum(oh1)
    counts = cnt0 + cnt1
    counts_ref[...] = counts

    pc = jnp.floor((counts + (BLK - 1)) / BLK) * BLK
    rj = lax.broadcasted_iota(jnp.int32, (E, E), 0)
    ck = lax.broadcasted_iota(jnp.int32, (E, E), 1)
    U = (rj < ck).astype(jnp.float32)
    off = lax.dot_general(pc, U, (((1,), (0,)), ((), ())),
                          preferred_element_type=jnp.float32)

    pos0 = jnp.sum(oh0 * (off + C0 - 1.0), axis=1, keepdims=True)
    pos1 = jnp.sum(oh1 * (off + cnt0 + C1 - 1.0), axis=1, keepdims=True)
    pos0_i = pos0.astype(jnp.int32)
    pos1_i = pos1.astype(jnp.int32)
    poscat_ref[0:NW // 2, :] = pos0_i.reshape(NW // 2, CHUNK)
    poscat_ref[NW // 2:NW, :] = pos1_i.reshape(NW // 2, CHUNK)
    pos0_ref[...] = pos0_i.reshape(NW, TCH)
    pos1_ref[...] = pos1_i.reshape(NW, TCH)
    wcat_ref[0:S, :] = jnp.broadcast_to(w0, (S, 128))
    wcat_ref[S:A, :] = jnp.broadcast_to(w1, (S, 128))

    ends = off + pc
    total = off[0, E - 1] + pc[0, E - 1]
    nact_ref[...] = jnp.full((1, 128), total / BLK, jnp.float32).astype(
        jnp.int32)
    bstart = (lax.broadcasted_iota(jnp.int32, (8, NB), 1)
              .astype(jnp.float32) * BLK)
    eid_raw = jnp.zeros((8, NB), jnp.float32)
    for e in range(E):
        eid_raw = eid_raw + (bstart >= ends[0, e]).astype(jnp.float32)
    e_iota = lax.broadcasted_iota(jnp.int32, (1, E), 1).astype(jnp.float32)
    last_e = jnp.max(jnp.where(pc > 0.0, e_iota, -1.0))
    eids = jnp.where(bstart < total, eid_raw, last_e)
    pad = jnp.zeros((8, 128 - NB), jnp.float32)
    eids_ref[...] = jnp.concatenate([eids, pad], axis=1).astype(jnp.int32)


_router = pl.pallas_call(
    _router_body,
    out_shape=(
        jax.ShapeDtypeStruct((NW, CHUNK), jnp.int32),
        jax.ShapeDtypeStruct((NW, TCH), jnp.int32),
        jax.ShapeDtypeStruct((NW, TCH), jnp.int32),
        jax.ShapeDtypeStruct((A, 128), jnp.float32),
        jax.ShapeDtypeStruct((8, 128), jnp.int32),
        jax.ShapeDtypeStruct((1, 128), jnp.int32),
        jax.ShapeDtypeStruct((1, E), jnp.float32),
    ),
)


def _scatter_body(x_hbm, wcat_hbm, pos_hbm, xg_hbm, sw_hbm, idx_v, rows_v,
                  wr_v, sem0, sem1):
    wid = lax.axis_index("s") * 2 + lax.axis_index("c")
    base = wid * CHUNK
    tbase = lax.rem(base, S)
    pltpu.sync_copy(pos_hbm.at[pl.ds(wid, 1)], idx_v)
    pltpu.sync_copy(x_hbm.at[pl.ds(tbase, CHUNK)], rows_v)
    pltpu.sync_copy(wcat_hbm.at[pl.ds(base, CHUNK)], wr_v)
    c0 = pltpu.async_copy(rows_v, xg_hbm.at[idx_v.at[0]], sem0)
    c1 = pltpu.async_copy(wr_v, sw_hbm.at[idx_v.at[0]], sem1)
    c0.wait()
    c1.wait()


@functools.cache
def _make_sc_kernels():
    mesh = plsc.VectorSubcoreMesh(core_axis_name="c", subcore_axis_name="s")
    scatter = pl.kernel(
        _scatter_body,
        out_type=(jax.ShapeDtypeStruct((G, D), jnp.float32),
                  jax.ShapeDtypeStruct((G, 128), jnp.float32)),
        mesh=mesh,
        scratch_types=[pltpu.VMEM((1, CHUNK), jnp.int32),
                       pltpu.VMEM((CHUNK, D), jnp.float32),
                       pltpu.VMEM((CHUNK, 128), jnp.float32),
                       pltpu.SemaphoreType.DMA,
                       pltpu.SemaphoreType.DMA],
    )
    combine = pl.kernel(
        _combine_body,
        out_type=jax.ShapeDtypeStruct((S, D), jnp.float32),
        mesh=mesh,
        scratch_types=[pltpu.VMEM((1, TCH), jnp.int32),
                       pltpu.VMEM((1, TCH), jnp.int32),
                       pltpu.VMEM((TCH, D), jnp.float32),
                       pltpu.VMEM((TCH, D), jnp.float32),
                       pltpu.SemaphoreType.DMA,
                       pltpu.SemaphoreType.DMA],
    )
    return scatter, combine


def _ffn_body(eids_ref, nact_ref, xg_ref, sw_ref, w1_ref, w3_ref, w2_ref,
              wg_ref, bg_ref, b1_ref, b3_ref, b2_ref, lg_ref, lb_ref,
              out_ref, acc_ref):
    f = pl.program_id(0)
    b = pl.program_id(1)

    @pl.when(b < nact_ref[0, 0])
    def _():
        xf = xg_ref[...]
        dn = (((1,), (1,)), ((), ()))
        h1 = lax.dot_general(xf, w1_ref[0], dn,
                             preferred_element_type=jnp.float32) + b1_ref[0]
        h3 = lax.dot_general(xf, w3_ref[0], dn,
                             preferred_element_type=jnp.float32) + b3_ref[0]
        h = (h1 * jax.nn.sigmoid(h1)) * (h3 * jax.nn.sigmoid(h3))
        part = lax.dot_general(h, w2_ref[0], dn,
                               preferred_element_type=jnp.float32)

        @pl.when(f == 0)
        def _():
            acc_ref[pl.ds(b * BLK, BLK), :] = part

        @pl.when(jnp.logical_and(f > 0, f < NF - 1))
        def _():
            acc_ref[pl.ds(b * BLK, BLK), :] += part

        @pl.when(f == NF - 1)
        def _():
            ff = acc_ref[pl.ds(b * BLK, BLK), :] + part + b2_ref[0]
            dgl = (jnp.sum(xf * wg_ref[0], axis=1, keepdims=True)
                   + bg_ref[0, :, :1])
            dg = jax.nn.sigmoid(dgl)
            wgt = sw_ref[:, :1]
            t = xf + dg * wgt * ff
            mu = jnp.mean(t, axis=1, keepdims=True)
            var = jnp.mean((t - mu) * (t - mu), axis=1, keepdims=True)
            y = (t - mu) / jnp.sqrt(var + 1e-5) * lg_ref[0] + lb_ref[0]
            out_ref[...] = y


_ffn = pl.pallas_call(
    _ffn_body,
    grid_spec=pltpu.PrefetchScalarGridSpec(
        num_scalar_prefetch=2,
        grid=(NF, NB),
        in_specs=[
            pl.BlockSpec((BLK, D), lambda f, b, er, nr: (b, 0)),
            pl.BlockSpec((BLK, 128), lambda f, b, er, nr: (b, 0)),
            pl.BlockSpec((1, FFS, D), lambda f, b, er, nr: (er[0, b], f, 0)),
            pl.BlockSpec((1, FFS, D), lambda f, b, er, nr: (er[0, b], f, 0)),
            pl.BlockSpec((1, D, FFS), lambda f, b, er, nr: (er[0, b], 0, f)),
            pl.BlockSpec((1, 1, D), lambda f, b, er, nr: (er[0, b], 0, 0)),
            pl.BlockSpec((1, 1, 128), lambda f, b, er, nr: (er[0, b], 0, 0)),
            pl.BlockSpec((1, 1, FFS), lambda f, b, er, nr: (er[0, b], 0, f)),
            pl.BlockSpec((1, 1, FFS), lambda f, b, er, nr: (er[0, b], 0, f)),
            pl.BlockSpec((1, 1, D), lambda f, b, er, nr: (er[0, b], 0, 0)),
            pl.BlockSpec((1, 1, D), lambda f, b, er, nr: (er[0, b], 0, 0)),
            pl.BlockSpec((1, 1, D), lambda f, b, er, nr: (er[0, b], 0, 0)),
        ],
        out_specs=pl.BlockSpec(
            (BLK, D), lambda f, b, er, nr: (jnp.where(f == NF - 1, b, NB), 0)),
        scratch_shapes=[pltpu.VMEM((G, D), jnp.float32)],
    ),
    out_shape=jax.ShapeDtypeStruct((G + BLK, D), jnp.float32),
)


def _combine_body(yg_hbm, pos0_hbm, pos1_hbm, out_hbm, i0, i1, r0, r1, sem0,
                  sem1):
    wid = lax.axis_index("s") * 2 + lax.axis_index("c")
    pltpu.sync_copy(pos0_hbm.at[pl.ds(wid, 1)], i0)
    pltpu.sync_copy(pos1_hbm.at[pl.ds(wid, 1)], i1)
    c0 = pltpu.async_copy(yg_hbm.at[i0.at[0]], r0, sem0)
    c1 = pltpu.async_copy(yg_hbm.at[i1.at[0]], r1, sem1)
    c0.wait()
    c1.wait()

    @pl.loop(0, TCH)
    def _(i):
        @pl.loop(0, D, step=16)
        def _(j):
            r0[i, pl.ds(j, 16)] = r0[i, pl.ds(j, 16)] + r1[i, pl.ds(j, 16)]

    pltpu.sync_copy(r0, out_hbm.at[pl.ds(wid * TCH, TCH)])


def kernel(x, router_W, router_b, Wg, bg, W1, b1, W2, b2, W3, b3, ln_g, ln_b):
    x2d = x.reshape(S, D)
    rb = router_b.reshape(1, E)

    poscat, pos0r, pos1r, wcat, eids, nact_arr, counts = _router(
        x2d, router_W, rb)
    expert_load = counts.reshape(E)

    scatter, combine = _make_sc_kernels()
    xg, sw = scatter(x2d, wcat, poscat)

    bgb = jnp.broadcast_to(bg.reshape(E, 1, 1), (E, 1, 128))
    yg = _ffn(eids, nact_arr, xg, sw, W1, W3, W2,
              Wg.reshape(E, 1, D), bgb,
              b1.reshape(E, 1, FF), b3.reshape(E, 1, FF),
              b2.reshape(E, 1, D), ln_g.reshape(E, 1, D), ln_b.reshape(E, 1, D))

    out2d = combine(yg, pos0r, pos1r)
    return out2d.reshape(1, S, D), expert_load


# speedup vs baseline: 1.1601x; 1.1601x over previous
import functools

import jax
import jax.numpy as jnp
from jax import lax
from jax.experimental import pallas as pl
from jax.experimental.pallas import tpu as pltpu
from jax.experimental.pallas import tpu_sc as plsc

S = 2048
D = 768
E = 8
FF = 4 * D
BLK = 256
NF = 2
FFS = FF // NF
G = 2 * S + E * BLK
NB = G // BLK
NW = 32
A = 2 * S
CHUNK = A // NW
TCH = S // NW


def _router_body(x_ref, rw_ref, rb_ref, poscat_ref, pos0_ref, pos1_ref,
                 wcat_ref, eids_ref, nact_ref, counts_ref):
    x = x_ref[...]
    logits = lax.dot_general(x, rw_ref[...], (((1,), (1,)), ((), ())),
                             preferred_element_type=jnp.float32)
    logits = logits + rb_ref[...]

    m = jnp.max(logits, axis=1, keepdims=True)
    ex = jnp.exp(logits - m)
    p = ex / jnp.sum(ex, axis=1, keepdims=True)

    lane = lax.broadcasted_iota(jnp.int32, (S, E), 1)
    m1 = jnp.max(p, axis=1, keepdims=True)
    i1 = jnp.min(jnp.where(p == m1, lane, E), axis=1, keepdims=True)
    p2 = jnp.where(lane == i1, -1.0, p)
    m2 = jnp.max(p2, axis=1, keepdims=True)
    i2 = jnp.min(jnp.where(p2 == m2, lane, E), axis=1, keepdims=True)

    e2 = jnp.exp(m2 - m1)
    w0 = 1.0 / (1.0 + e2)
    w1 = e2 / (1.0 + e2)

    oh0 = (lane == i1).astype(jnp.float32)
    oh1 = (lane == i2).astype(jnp.float32)

    CH = 128
    r_i = lax.broadcasted_iota(jnp.int32, (CH, CH), 0)
    c_i = lax.broadcasted_iota(jnp.int32, (CH, CH), 1)
    L = (r_i >= c_i).astype(jnp.float32)

    def chunked_cumsum(oh):
        outs = []
        base = jnp.zeros((1, E), jnp.float32)
        for c in range(S // CH):
            blk = oh[c * CH:(c + 1) * CH, :]
            cc = lax.dot_general(L, blk, (((1,), (0,)), ((), ())),
                                 preferred_element_type=jnp.float32) + base
            outs.append(cc)
            base = cc[CH - 1:CH, :]
        return jnp.concatenate(outs, axis=0), base

    C0, cnt0 = chunked_cumsum(oh0)
    C1, cnt1 = chunked_cumsum(oh1)
    counts = cnt0 + cnt1
    counts_ref[...] = counts

    pc = jnp.floor((counts + (BLK - 1)) / BLK) * BLK
    rj = lax.broadcasted_iota(jnp.int32, (E, E), 0)
    ck = lax.broadcasted_iota(jnp.int32, (E, E), 1)
    U = (rj < ck).astype(jnp.float32)
    off = lax.dot_general(pc, U, (((1,), (0,)), ((), ())),
                          preferred_element_type=jnp.float32)

    pos0 = jnp.sum(oh0 * (off + C0 - 1.0), axis=1, keepdims=True)
    pos1 = jnp.sum(oh1 * (off + cnt0 + C1 - 1.0), axis=1, keepdims=True)
    pos0_i = pos0.astype(jnp.int32)
    pos1_i = pos1.astype(jnp.int32)
    poscat_ref[0:NW // 2, :] = pos0_i.reshape(NW // 2, CHUNK)
    poscat_ref[NW // 2:NW, :] = pos1_i.reshape(NW // 2, CHUNK)
    pos0_ref[...] = pos0_i.reshape(NW, TCH)
    pos1_ref[...] = pos1_i.reshape(NW, TCH)
    wcat_ref[0:S, :] = jnp.broadcast_to(w0, (S, 128))
    wcat_ref[S:A, :] = jnp.broadcast_to(w1, (S, 128))

    ends = off + pc
    total = off[0, E - 1] + pc[0, E - 1]
    nact_ref[...] = jnp.full((1, 128), total / BLK, jnp.float32).astype(
        jnp.int32)
    bstart = (lax.broadcasted_iota(jnp.int32, (8, NB), 1)
              .astype(jnp.float32) * BLK)
    eid_raw = jnp.zeros((8, NB), jnp.float32)
    for e in range(E):
        eid_raw = eid_raw + (bstart >= ends[0, e]).astype(jnp.float32)
    e_iota = lax.broadcasted_iota(jnp.int32, (1, E), 1).astype(jnp.float32)
    last_e = jnp.max(jnp.where(pc > 0.0, e_iota, -1.0))
    eids = jnp.where(bstart < total, eid_raw, last_e)
    pad = jnp.zeros((8, 128 - NB), jnp.float32)
    eids_ref[...] = jnp.concatenate([eids, pad], axis=1).astype(jnp.int32)


_router = pl.pallas_call(
    _router_body,
    out_shape=(
        jax.ShapeDtypeStruct((NW, CHUNK), jnp.int32),
        jax.ShapeDtypeStruct((NW, TCH), jnp.int32),
        jax.ShapeDtypeStruct((NW, TCH), jnp.int32),
        jax.ShapeDtypeStruct((A, 128), jnp.float32),
        jax.ShapeDtypeStruct((8, 128), jnp.int32),
        jax.ShapeDtypeStruct((1, 128), jnp.int32),
        jax.ShapeDtypeStruct((1, E), jnp.float32),
    ),
)


def _scatter_body(x_hbm, wcat_hbm, pos_hbm, xg_hbm, sw_hbm, idx_v, rows_v,
                  wr_v, sem0, sem1):
    wid = lax.axis_index("s") * 2 + lax.axis_index("c")
    base = wid * CHUNK
    tbase = lax.rem(base, S)
    pltpu.sync_copy(pos_hbm.at[pl.ds(wid, 1)], idx_v)
    pltpu.sync_copy(x_hbm.at[pl.ds(tbase, CHUNK)], rows_v)
    pltpu.sync_copy(wcat_hbm.at[pl.ds(base, CHUNK)], wr_v)
    c0 = pltpu.async_copy(rows_v, xg_hbm.at[idx_v.at[0]], sem0)
    c1 = pltpu.async_copy(wr_v, sw_hbm.at[idx_v.at[0]], sem1)
    c0.wait()
    c1.wait()


@functools.cache
def _make_sc_kernels():
    mesh = plsc.VectorSubcoreMesh(core_axis_name="c", subcore_axis_name="s")
    scatter = pl.kernel(
        _scatter_body,
        out_type=(jax.ShapeDtypeStruct((G, D), jnp.float32),
                  jax.ShapeDtypeStruct((G, 128), jnp.float32)),
        mesh=mesh,
        scratch_types=[pltpu.VMEM((1, CHUNK), jnp.int32),
                       pltpu.VMEM((CHUNK, D), jnp.float32),
                       pltpu.VMEM((CHUNK, 128), jnp.float32),
                       pltpu.SemaphoreType.DMA,
                       pltpu.SemaphoreType.DMA],
    )
    combine = pl.kernel(
        _combine_body,
        out_type=jax.ShapeDtypeStruct((S, D), jnp.float32),
        mesh=mesh,
        scratch_types=[pltpu.VMEM((1, TCH), jnp.int32),
                       pltpu.VMEM((1, TCH), jnp.int32),
                       pltpu.VMEM((TCH, D), jnp.float32),
                       pltpu.VMEM((TCH, D), jnp.float32),
                       pltpu.SemaphoreType.DMA,
                       pltpu.SemaphoreType.DMA],
    )
    return scatter, combine


def _ffn_body(eids_ref, nact_ref, xg_ref, sw_ref, w1_ref, w3_ref, w2_ref,
              wg_ref, bg_ref, b1_ref, b3_ref, b2_ref, lg_ref, lb_ref,
              out_ref, acc_ref):
    f = pl.program_id(0)
    b = pl.program_id(1)

    @pl.when(b < nact_ref[0, 0])
    def _():
        xf = xg_ref[...]
        dn = (((1,), (1,)), ((), ()))
        h1 = lax.dot_general(xf, w1_ref[0], dn,
                             preferred_element_type=jnp.float32) + b1_ref[0]
        h3 = lax.dot_general(xf, w3_ref[0], dn,
                             preferred_element_type=jnp.float32) + b3_ref[0]
        h = (h1 * jax.nn.sigmoid(h1)) * (h3 * jax.nn.sigmoid(h3))
        part = lax.dot_general(h, w2_ref[0], dn,
                               preferred_element_type=jnp.float32)

        @pl.when(f == 0)
        def _():
            acc_ref[pl.ds(b * BLK, BLK), :] = part

        @pl.when(jnp.logical_and(f > 0, f < NF - 1))
        def _():
            acc_ref[pl.ds(b * BLK, BLK), :] += part

        @pl.when(f == NF - 1)
        def _():
            ff = acc_ref[pl.ds(b * BLK, BLK), :] + part + b2_ref[0]
            dgl = (jnp.sum(xf * wg_ref[0], axis=1, keepdims=True)
                   + bg_ref[0, :, :1])
            dg = jax.nn.sigmoid(dgl)
            wgt = sw_ref[:, :1]
            t = xf + dg * wgt * ff
            mu = jnp.mean(t, axis=1, keepdims=True)
            var = jnp.mean((t - mu) * (t - mu), axis=1, keepdims=True)
            y = (t - mu) / jnp.sqrt(var + 1e-5) * lg_ref[0] + lb_ref[0]
            out_ref[...] = y


_ffn = pl.pallas_call(
    _ffn_body,
    grid_spec=pltpu.PrefetchScalarGridSpec(
        num_scalar_prefetch=2,
        grid=(NF, NB),
        in_specs=[
            pl.BlockSpec((BLK, D), lambda f, b, er, nr: (b, 0)),
            pl.BlockSpec((BLK, 128), lambda f, b, er, nr: (b, 0)),
            pl.BlockSpec((1, FFS, D), lambda f, b, er, nr: (er[0, b], f, 0)),
            pl.BlockSpec((1, FFS, D), lambda f, b, er, nr: (er[0, b], f, 0)),
            pl.BlockSpec((1, D, FFS), lambda f, b, er, nr: (er[0, b], 0, f)),
            pl.BlockSpec((1, 1, D), lambda f, b, er, nr: (er[0, b], 0, 0)),
            pl.BlockSpec((1, 1, 128), lambda f, b, er, nr: (er[0, b], 0, 0)),
            pl.BlockSpec((1, 1, FFS), lambda f, b, er, nr: (er[0, b], 0, f)),
            pl.BlockSpec((1, 1, FFS), lambda f, b, er, nr: (er[0, b], 0, f)),
            pl.BlockSpec((1, 1, D), lambda f, b, er, nr: (er[0, b], 0, 0)),
            pl.BlockSpec((1, 1, D), lambda f, b, er, nr: (er[0, b], 0, 0)),
            pl.BlockSpec((1, 1, D), lambda f, b, er, nr: (er[0, b], 0, 0)),
        ],
        out_specs=pl.BlockSpec(
            (BLK, D), lambda f, b, er, nr: (jnp.where(f == NF - 1, b, NB), 0)),
        scratch_shapes=[pltpu.VMEM((G, D), jnp.float32)],
    ),
    out_shape=jax.ShapeDtypeStruct((G + BLK, D), jnp.float32),
)


def _combine_body(yg_hbm, pos0_hbm, pos1_hbm, out_hbm, i0, i1, r0, r1, sem0,
                  sem1):
    wid = lax.axis_index("s") * 2 + lax.axis_index("c")
    pltpu.sync_copy(pos0_hbm.at[pl.ds(wid, 1)], i0)
    pltpu.sync_copy(pos1_hbm.at[pl.ds(wid, 1)], i1)
    c0 = pltpu.async_copy(yg_hbm.at[i0.at[0]], r0, sem0)
    c1 = pltpu.async_copy(yg_hbm.at[i1.at[0]], r1, sem1)
    c0.wait()
    c1.wait()

    @pl.loop(0, TCH)
    def _(i):
        @pl.loop(0, D, step=16)
        def _(j):
            r0[i, pl.ds(j, 16)] = r0[i, pl.ds(j, 16)] + r1[i, pl.ds(j, 16)]

    pltpu.sync_copy(r0, out_hbm.at[pl.ds(wid * TCH, TCH)])


def kernel(x, router_W, router_b, Wg, bg, W1, b1, W2, b2, W3, b3, ln_g, ln_b):
    x2d = x.reshape(S, D)
    rb = router_b.reshape(1, E)

    poscat, pos0r, pos1r, wcat, eids, nact_arr, counts = _router(
        x2d, router_W, rb)
    expert_load = counts.reshape(E)

    scatter, combine = _make_sc_kernels()
    xg, sw = scatter(x2d, wcat, poscat)

    bgb = jnp.broadcast_to(bg.reshape(E, 1, 1), (E, 1, 128))
    yg = _ffn(eids, nact_arr, xg, sw, W1, W3, W2,
              Wg.reshape(E, 1, D), bgb,
              b1.reshape(E, 1, FF), b3.reshape(E, 1, FF),
              b2.reshape(E, 1, D), ln_g.reshape(E, 1, D), ln_b.reshape(E, 1, D))

    out2d = combine(yg, pos0r, pos1r)
    return out2d.reshape(1, S, D), expert_load
